# baseline (speedup 1.0000x reference)
import jax
import jax.numpy as jnp
from jax import lax
from jax.experimental import pallas as pl
from jax.experimental.pallas import tpu as pltpu
from jax.experimental.pallas import tpu_sc as plsc

_NC = 2
_NS = 16
_NW = _NC * _NS

_BATCH = 16384
_FIELDS = 26
_DIM = 32
_BROWS_PER_W = _BATCH // _NW
_CHUNK_BROWS = 64
_N_CHUNKS = _BROWS_PER_W // _CHUNK_BROWS


def _gather_body(tbl_hbm, idx_hbm, out_hbm, idx_v, buf0, buf1, gsem0, gsem1,
                 ssem0, ssem1):
    wid = lax.axis_index("s") * _NC + lax.axis_index("c")
    brow0 = wid * _BROWS_PER_W
    pltpu.sync_copy(idx_hbm.at[pl.ds(brow0, _BROWS_PER_W)], idx_v)

    bufs = (buf0, buf1)
    gsems = (gsem0, gsem1)
    ssems = (ssem0, ssem1)

    def fire_gather(i):
        p = i % 2

        def enqueue(j, carry):
            pltpu.async_copy(
                tbl_hbm.at[idx_v.at[i * _CHUNK_BROWS + j]],
                bufs[p].at[j], gsems[p])
            return carry

        lax.fori_loop(0, _CHUNK_BROWS, enqueue, 0)

    def drain_gather(i):
        p = i % 2
        pltpu.make_async_copy(
            out_hbm.at[pl.ds(brow0 + i * _CHUNK_BROWS, _CHUNK_BROWS)],
            bufs[p], gsems[p]).wait()

    def fire_store(i):
        p = i % 2
        return pltpu.async_copy(
            bufs[p], out_hbm.at[pl.ds(brow0 + i * _CHUNK_BROWS, _CHUNK_BROWS)],
            ssems[p])

    stores = [None] * _N_CHUNKS
    fire_gather(0)
    for i in range(_N_CHUNKS):
        drain_gather(i)
        if i >= 1:
            stores[i - 1].wait()
        if i + 1 < _N_CHUNKS:
            fire_gather(i + 1)
        stores[i] = fire_store(i)
    stores[_N_CHUNKS - 1].wait()


_gather = pl.kernel(
    _gather_body,
    out_type=jax.ShapeDtypeStruct((_BATCH, _FIELDS, _DIM), jnp.float32),
    mesh=plsc.VectorSubcoreMesh(
        core_axis_name="c", subcore_axis_name="s",
        num_cores=_NC, num_subcores=_NS,
    ),
    scratch_types=[
        pltpu.VMEM((_BROWS_PER_W, _FIELDS), jnp.int32),
        pltpu.VMEM((_CHUNK_BROWS, _FIELDS, _DIM), jnp.float32),
        pltpu.VMEM((_CHUNK_BROWS, _FIELDS, _DIM), jnp.float32),
        pltpu.SemaphoreType.DMA,
        pltpu.SemaphoreType.DMA,
        pltpu.SemaphoreType.DMA,
        pltpu.SemaphoreType.DMA,
    ],
    compiler_params=pltpu.CompilerParams(use_tc_tiling_on_sc=False),
)


def kernel(input_tensor, weight):
    return _gather(weight, input_tensor.astype(jnp.int32))


# speedup vs baseline: 1.0035x; 1.0035x over previous
import jax
import jax.numpy as jnp
from jax import lax
from jax.experimental import pallas as pl
from jax.experimental.pallas import tpu as pltpu
from jax.experimental.pallas import tpu_sc as plsc

_NC = 2
_NS = 16
_NW = _NC * _NS

_BATCH = 16384
_FIELDS = 26
_DIM = 32
_BROWS_PER_W = _BATCH // _NW
_CHUNK_BROWS = 64
_N_CHUNKS = _BROWS_PER_W // _CHUNK_BROWS
_CHUNK_ROWS = _CHUNK_BROWS * _FIELDS


def _gather_body(tbl_hbm, idx_hbm, out_hbm, idx_v, buf0, buf1, gsem0, gsem1,
                 ssem0, ssem1):
    wid = lax.axis_index("s") * _NC + lax.axis_index("c")
    brow0 = wid * _BROWS_PER_W
    row0 = brow0 * _FIELDS
    pltpu.sync_copy(idx_hbm.at[pl.ds(brow0, _BROWS_PER_W)], idx_v)

    bufs = (buf0, buf1)
    gsems = (gsem0, gsem1)
    ssems = (ssem0, ssem1)

    def fire_gather(i):
        p = i % 2

        def enqueue(j, carry):
            pltpu.async_copy(
                tbl_hbm.at[idx_v.at[i * _CHUNK_BROWS + j]],
                bufs[p].at[pl.ds(j * _FIELDS, _FIELDS)], gsems[p])
            return carry

        lax.fori_loop(0, _CHUNK_BROWS, enqueue, 0)

    def drain_gather(i):
        p = i % 2
        pltpu.make_async_copy(
            out_hbm.at[pl.ds(row0 + i * _CHUNK_ROWS, _CHUNK_ROWS)],
            bufs[p], gsems[p]).wait()

    def fire_store(i):
        p = i % 2
        return pltpu.async_copy(
            bufs[p], out_hbm.at[pl.ds(row0 + i * _CHUNK_ROWS, _CHUNK_ROWS)],
            ssems[p])

    stores = [None] * _N_CHUNKS
    fire_gather(0)
    for i in range(_N_CHUNKS):
        drain_gather(i)
        if i >= 1:
            stores[i - 1].wait()
        if i + 1 < _N_CHUNKS:
            fire_gather(i + 1)
        stores[i] = fire_store(i)
    stores[_N_CHUNKS - 1].wait()


_gather = pl.kernel(
    _gather_body,
    out_type=jax.ShapeDtypeStruct((_BATCH * _FIELDS, _DIM), jnp.float32),
    mesh=plsc.VectorSubcoreMesh(
        core_axis_name="c", subcore_axis_name="s",
        num_cores=_NC, num_subcores=_NS,
    ),
    scratch_types=[
        pltpu.VMEM((_BROWS_PER_W, _FIELDS), jnp.int32),
        pltpu.VMEM((_CHUNK_ROWS, _DIM), jnp.float32),
        pltpu.VMEM((_CHUNK_ROWS, _DIM), jnp.float32),
        pltpu.SemaphoreType.DMA,
        pltpu.SemaphoreType.DMA,
        pltpu.SemaphoreType.DMA,
        pltpu.SemaphoreType.DMA,
    ],
    compiler_params=pltpu.CompilerParams(use_tc_tiling_on_sc=False),
)


def kernel(input_tensor, weight):
    out = _gather(weight, input_tensor.astype(jnp.int32))
    return out.reshape(_BATCH, _FIELDS, _DIM)
